# Initial kernel scaffold; baseline (speedup 1.0000x reference)
import jax
import jax.numpy as jnp
from jax.experimental import pallas as pl


def kernel(queries, keys_db, lengths, m_lengths):
    raise NotImplementedError("write your pallas kernel here")


# speedup vs baseline: 2.7345x; 2.7345x over previous
import functools

import jax
import jax.numpy as jnp
from jax.experimental import pallas as pl
from jax.experimental.pallas import tpu as pltpu

_KINEMATIC_COEF = 0.1
_NEG_INF = float("-inf")
_BIG_I32 = 2**31 - 1


def _retrieve_body(q_ref, kdb_ref, ql_ref, ml_ref, vals_ref, idx_ref,
                   qn_ref, r1v_ref, r1i_ref, r2v_ref, r2i_ref,
                   *, block_k, num_keys, num_blocks):
    k = pl.program_id(0)

    @pl.when(k == 0)
    def _init():
        q = q_ref[...]
        qnorm = jnp.sqrt(jnp.sum(q * q, axis=1, keepdims=True))
        qn_ref[...] = q / jnp.maximum(qnorm, 1e-8)
        r1v_ref[...] = jnp.full(r1v_ref.shape, _NEG_INF, jnp.float32)
        r2v_ref[...] = jnp.full(r2v_ref.shape, _NEG_INF, jnp.float32)
        r1i_ref[...] = jnp.zeros(r1i_ref.shape, jnp.int32)
        r2i_ref[...] = jnp.zeros(r2i_ref.shape, jnp.int32)

    kb = kdb_ref[...]
    knorm = jnp.sqrt(jnp.sum(kb * kb, axis=1, keepdims=True))
    kn = kb / jnp.maximum(knorm, 1e-8)

    semantic = jax.lax.dot_general(
        qn_ref[...], kn,
        dimension_numbers=(((1,), (1,)), ((), ())),
        preferred_element_type=jnp.float32,
    )

    ml = ml_ref[...]
    ql = ql_ref[...]
    denom = jnp.maximum(jnp.maximum(ml, ql), 1.0)
    rel = jnp.abs(ml - ql) / denom
    score = semantic * jnp.exp(rel * (-_KINEMATIC_COEF))

    q_dim = score.shape[0]
    gidx = (jax.lax.broadcasted_iota(jnp.int32, (q_dim, block_k), 1)
            + k * block_k)
    score = jnp.where(gidx < num_keys, score, _NEG_INF)

    m1v = jnp.max(score, axis=1, keepdims=True)
    m1i = jnp.min(jnp.where(score == m1v, gidx, _BIG_I32), axis=1,
                  keepdims=True)
    masked = jnp.where(gidx == m1i, _NEG_INF, score)
    m2v = jnp.max(masked, axis=1, keepdims=True)
    m2i = jnp.min(jnp.where(masked == m2v, gidx, _BIG_I32), axis=1,
                  keepdims=True)

    r1v, r1i = r1v_ref[...], r1i_ref[...]
    r2v, r2i = r2v_ref[...], r2i_ref[...]
    first_run = r1v >= m1v
    n1v = jnp.where(first_run, r1v, m1v)
    n1i = jnp.where(first_run, r1i, m1i)
    cr = r2v >= m1v
    cb = r1v >= m2v
    n2v = jnp.where(first_run, jnp.where(cr, r2v, m1v),
                    jnp.where(cb, r1v, m2v))
    n2i = jnp.where(first_run, jnp.where(cr, r2i, m1i),
                    jnp.where(cb, r1i, m2i))
    r1v_ref[...], r1i_ref[...] = n1v, n1i
    r2v_ref[...], r2i_ref[...] = n2v, n2i

    @pl.when(k == num_blocks - 1)
    def _finish():
        vals_ref[...] = jnp.concatenate([n1v, n2v], axis=1)
        idx_ref[...] = jnp.concatenate([n1i, n2i], axis=1)


def kernel(queries, keys_db, lengths, m_lengths):
    q_dim, d = queries.shape
    num_keys = keys_db.shape[0]
    block_k = 1024
    num_blocks = pl.cdiv(num_keys, block_k)

    ql = lengths.astype(jnp.float32).reshape(q_dim, 1)
    ml = m_lengths.astype(jnp.float32).reshape(1, num_keys)

    body = functools.partial(_retrieve_body, block_k=block_k,
                             num_keys=num_keys, num_blocks=num_blocks)

    vals, idx = pl.pallas_call(
        body,
        grid=(num_blocks,),
        in_specs=[
            pl.BlockSpec((q_dim, d), lambda k: (0, 0)),
            pl.BlockSpec((block_k, d), lambda k: (k, 0)),
            pl.BlockSpec((q_dim, 1), lambda k: (0, 0)),
            pl.BlockSpec((1, block_k), lambda k: (0, k)),
        ],
        out_specs=[
            pl.BlockSpec((q_dim, 2), lambda k: (0, 0)),
            pl.BlockSpec((q_dim, 2), lambda k: (0, 0)),
        ],
        out_shape=[
            jax.ShapeDtypeStruct((q_dim, 2), jnp.float32),
            jax.ShapeDtypeStruct((q_dim, 2), jnp.int32),
        ],
        scratch_shapes=[
            pltpu.VMEM((q_dim, d), jnp.float32),
            pltpu.VMEM((q_dim, 1), jnp.float32),
            pltpu.VMEM((q_dim, 1), jnp.int32),
            pltpu.VMEM((q_dim, 1), jnp.float32),
            pltpu.VMEM((q_dim, 1), jnp.int32),
        ],
        compiler_params=pltpu.CompilerParams(
            dimension_semantics=("arbitrary",),
        ),
    )(queries, keys_db, ql, ml)
    return vals, idx
